# baseline (speedup 1.0000x reference)
import jax
import jax.numpy as jnp
from jax import lax
from jax.experimental import pallas as pl
from jax.experimental.pallas import tpu as pltpu
from jax.experimental.pallas import tpu_sc as plsc

H = 8
C = 16
NEG_SLOPE = 0.2

NC = 2
NS = 16
NW = NC * NS
K = 40
NBUF = 5
NBI = 5
GL = 2
IXL = 3
W144 = H * C + 16


def _mm_body(x_ref, w_ref, s_ref, hx_ref, ad_ref):
    hb = jnp.dot(x_ref[...], w_ref[...], preferred_element_type=jnp.float32)
    a4 = jnp.dot(hb, s_ref[...], preferred_element_type=jnp.float32)
    hx_ref[...] = jnp.concatenate([hb, a4[:, :16]], axis=1)
    ad_ref[...] = a4[:, 16:]


def _epi_body(r0_ref, r1_ref, p_ref, b_ref, o_ref):
    r = r0_ref[0] + r1_ref[0]
    d = r[:, H * C:H * C + H]
    den = jnp.dot(d, p_ref[...], preferred_element_type=jnp.float32) + 1e-16
    v = r[:, :H * C] / den + b_ref[...]
    o_ref[...] = jnp.where(v > 0, v, jnp.exp(jnp.minimum(v, 0.0)) - 1.0)


def _rowcopy_split(s, n_rows, copy_fn):
    n0 = (n_rows // NS) & ~7
    rem = n_rows - n0 * NS
    copy_fn(s * n0, n0)
    if rem:
        @pl.when(s == NS - 1)
        def _():
            copy_fn(NS * n0, rem)


def _edge_body(hx_hbm, ad_hbm, sd2_hbm,
               acc_out,
               sdv, rd, hr, acc_sh, isem, gsem, ssem):
    c = lax.axis_index("c")
    s = lax.axis_index("s")
    wid = c * NS + s
    nch = sd2_hbm.shape[1]

    def zrow(i, acc):
        for q in range(W144 // 16):
            hr[0][i, pl.ds(q * 16, 16)] = jnp.zeros((16,), jnp.float32)
        return acc
    lax.fori_loop(0, K, zrow, 0)

    def zero_acc(r0, n):
        for q in range(0, n, K):
            m = min(K, n - q)
            pltpu.async_copy(hr[0].at[pl.ds(0, m)],
                             acc_sh.at[pl.ds(r0 + q, m)], isem[0])

    def drain_acc(r0, n):
        for q in range(0, n, K):
            m = min(K, n - q)
            pltpu.make_async_copy(hr[0].at[pl.ds(0, m)],
                                  acc_sh.at[pl.ds(r0 + q, m)], isem[0]).wait()

    _rowcopy_split(s, acc_sh.shape[0], zero_acc)
    _rowcopy_split(s, acc_sh.shape[0], drain_acc)
    plsc.subcore_barrier()

    def issue_idx(j, bi):
        pltpu.async_copy(sd2_hbm.at[wid, j], sdv[bi], isem[bi])

    def wait_idx(j, bi):
        pltpu.make_async_copy(sd2_hbm.at[wid, j], sdv[bi], isem[bi]).wait()

    def issue_gathers(b, bi):
        pltpu.async_copy(hx_hbm.at[sdv[bi].at[0]], hr[b], gsem[b])
        pltpu.async_copy(ad_hbm.at[sdv[bi].at[1]], rd[b], gsem[b])

    def wait_gathers(b, bi):
        pltpu.make_async_copy(hx_hbm.at[sdv[bi].at[0]], hr[b], gsem[b]).wait()
        pltpu.make_async_copy(ad_hbm.at[sdv[bi].at[1]], rd[b], gsem[b]).wait()

    def issue_scatter(b, bi):
        pltpu.async_copy(hr[b], acc_sh.at[sdv[bi].at[1]], ssem[b], add=True)

    def wait_scatter(b, bi):
        pltpu.make_async_copy(hr[b], acc_sh.at[sdv[bi].at[1]], ssem[b]).wait()

    def compute(b):
        def edge(i, acc):
            a = hr[b][i, pl.ds(H * C, 16)] + rd[b][i, :]
            se = jnp.exp(jnp.maximum(a, NEG_SLOPE * a))
            hr[b][i, pl.ds(H * C, 16)] = se
            for hd in range(H):
                coefv = lax.gather(
                    se, jnp.full((16, 1), hd, dtype=jnp.int32),
                    lax.GatherDimensionNumbers(offset_dims=(),
                                               collapsed_slice_dims=(0,),
                                               start_index_map=(0,)),
                    slice_sizes=(1,),
                    mode=lax.GatherScatterMode.PROMISE_IN_BOUNDS)
                hr[b][i, pl.ds(hd * 16, 16)] = hr[b][i, pl.ds(hd * 16, 16)] * coefv
            return acc
        lax.fori_loop(0, K, edge, 0)

    for j in range(IXL):
        issue_idx(j, j % NBI)
    for j in range(GL):
        wait_idx(j, j % NBI)
        issue_gathers(j % NBUF, j % NBI)

    UNROLL = 5

    def group(i, acc):
        for t in range(UNROLL):
            j = i * UNROLL + t
            b = t % NBUF
            bi = t % NBI
            if t >= 2:
                wait_scatter((t - 2) % NBUF, (t - 2) % NBI)
            else:
                @pl.when(i >= 1)
                def _():
                    wait_scatter((t - 2) % NBUF, (t - 2) % NBI)

            @pl.when(j + IXL < nch)
            def _():
                issue_idx(j + IXL, (t + IXL) % NBI)

            @pl.when(j + GL < nch)
            def _():
                wait_idx(j + GL, (t + GL) % NBI)
                issue_gathers((t + GL) % NBUF, (t + GL) % NBI)
            wait_gathers(b, bi)
            compute(b)
            issue_scatter(b, bi)
        return acc
    ngroups = nch // UNROLL
    lax.fori_loop(0, ngroups, group, 0)

    for j in range(ngroups * UNROLL, nch):
        b = j % NBUF
        bi = j % NBI
        wait_scatter((j - 2) % NBUF, (j - 2) % NBI)
        if j + IXL < nch:
            issue_idx(j + IXL, (j + IXL) % NBI)
        if j + GL < nch:
            wait_idx(j + GL, (j + GL) % NBI)
            issue_gathers((j + GL) % NBUF, (j + GL) % NBI)
        wait_gathers(b, bi)
        compute(b)
        issue_scatter(b, bi)
    for j in range(nch - 2, nch):
        wait_scatter(j % NBUF, j % NBI)

    plsc.subcore_barrier()
    _rowcopy_split(s, acc_sh.shape[0], lambda r0, n: pltpu.sync_copy(
        acc_sh.at[pl.ds(r0, n)], acc_out.at[c, pl.ds(r0, n)]))


def kernel(x, edge_index, W, att_src, att_dst, bias):
    N, D = x.shape
    E = edge_index.shape[1]
    HC = H * C
    nch = E // (NW * K)
    sd2 = jnp.stack([edge_index[0].astype(jnp.int32).reshape(NW, nch, K),
                     edge_index[1].astype(jnp.int32).reshape(NW, nch, K)],
                    axis=2)

    asf = att_src.reshape(HC)
    adf = att_dst.reshape(HC)
    head_of = (jnp.arange(HC, dtype=jnp.int32) // C)
    M = (head_of[:, None] == jnp.arange(H, dtype=jnp.int32)[None, :]).astype(jnp.float32)
    S = jnp.concatenate([M * asf[:, None], M * adf[:, None],
                         M * adf[:, None], M * asf[:, None]], axis=1)
    P = M.T

    BLK = 1000
    grid = (N // BLK,)

    hx, ad = pl.pallas_call(
        _mm_body,
        grid=grid,
        in_specs=[pl.BlockSpec((BLK, D), lambda i: (i, 0)),
                  pl.BlockSpec((D, HC), lambda i: (0, 0)),
                  pl.BlockSpec((D, 2 * C), lambda i: (0, 0))],
        out_specs=[pl.BlockSpec((BLK, W144), lambda i: (i, 0)),
                   pl.BlockSpec((BLK, C), lambda i: (i, 0))],
        out_shape=[jax.ShapeDtypeStruct((N, W144), jnp.float32),
                   jax.ShapeDtypeStruct((N, C), jnp.float32)],
    )(x, W, S)

    mesh = plsc.VectorSubcoreMesh(core_axis_name="c", subcore_axis_name="s",
                                  num_cores=NC, num_subcores=NS)

    acc = pl.kernel(
        _edge_body,
        out_type=jax.ShapeDtypeStruct((NC, N, W144), jnp.float32),
        mesh=mesh,
        compiler_params=pltpu.CompilerParams(use_tc_tiling_on_sc=False),
        scratch_types=[
            [pltpu.VMEM((2, K), jnp.int32)] * NBI,
            [pltpu.VMEM((K, 16), jnp.float32)] * NBUF,
            [pltpu.VMEM((K, W144), jnp.float32)] * NBUF,
            pltpu.VMEM_SHARED((N, W144), jnp.float32),
            [pltpu.SemaphoreType.DMA] * NBI,
            [pltpu.SemaphoreType.DMA] * NBUF,
            [pltpu.SemaphoreType.DMA] * NBUF,
        ],
    )(hx, ad, sd2)

    bias2d = bias.reshape(1, HC)
    out = pl.pallas_call(
        _epi_body,
        grid=grid,
        in_specs=[pl.BlockSpec((1, BLK, W144), lambda i: (0, i, 0)),
                  pl.BlockSpec((1, BLK, W144), lambda i: (1, i, 0)),
                  pl.BlockSpec((H, HC), lambda i: (0, 0)),
                  pl.BlockSpec((1, HC), lambda i: (0, 0))],
        out_specs=pl.BlockSpec((BLK, HC), lambda i: (i, 0)),
        out_shape=jax.ShapeDtypeStruct((N, HC), jnp.float32),
    )(acc, acc, P, bias2d)
    return out


# speedup vs baseline: 1.1220x; 1.1220x over previous
import jax
import jax.numpy as jnp
from jax import lax
from jax.experimental import pallas as pl
from jax.experimental.pallas import tpu as pltpu
from jax.experimental.pallas import tpu_sc as plsc

H = 8
C = 16
NEG_SLOPE = 0.2

NC = 2
NS = 16
NW = NC * NS
K = 40
NBUF = 5


def _mm_body(x_ref, w_ref, s_ref, h_ref, a4_ref):
    hb = jnp.dot(x_ref[...], w_ref[...], preferred_element_type=jnp.float32)
    h_ref[...] = hb
    a4_ref[...] = jnp.dot(hb, s_ref[...], preferred_element_type=jnp.float32)


def _epi_body(r0_ref, r1_ref, d0_ref, d1_ref, p_ref, b_ref, o_ref):
    d = d0_ref[0] + d1_ref[0]
    den = jnp.dot(d, p_ref[...], preferred_element_type=jnp.float32) + 1e-16
    v = (r0_ref[0] + r1_ref[0]) / den + b_ref[...]
    o_ref[...] = jnp.where(v > 0, v, jnp.exp(jnp.minimum(v, 0.0)) - 1.0)


def _rowcopy_split(s, n_rows, copy_fn):
    n0 = (n_rows // NS) & ~7
    rem = n_rows - n0 * NS
    copy_fn(s * n0, n0)
    if rem:
        @pl.when(s == NS - 1)
        def _():
            copy_fn(NS * n0, rem)


def _edge_body(as_hbm, ad_hbm, h_hbm, src2_hbm, dst2_hbm,
               dpart, raw,
               srcv, dstv, rs, rd, hr, denom_sh, out_sh, isem, gsem, ssem):
    c = lax.axis_index("c")
    s = lax.axis_index("s")
    wid = c * NS + s
    nch = src2_hbm.shape[1]

    def zrow(i, acc):
        rs[0][i, :] = jnp.zeros((16,), jnp.float32)
        for q in range(H):
            hr[0][i, pl.ds(q * 16, 16)] = jnp.zeros((16,), jnp.float32)
        return acc
    lax.fori_loop(0, K, zrow, 0)

    def zero_acc(zbuf, acc_sh, r0, n):
        for q in range(0, n, K):
            m = min(K, n - q)
            pltpu.async_copy(zbuf.at[pl.ds(0, m)],
                             acc_sh.at[pl.ds(r0 + q, m)], isem[0])
    def zero16(r0, n):
        zero_acc(rs[0], denom_sh, r0, n)

    def zero128(r0, n):
        zero_acc(hr[0], out_sh, r0, n)

    def drain16(r0, n):
        for q in range(0, n, K):
            m = min(K, n - q)
            pltpu.make_async_copy(rs[0].at[pl.ds(0, m)],
                                  denom_sh.at[pl.ds(r0 + q, m)], isem[0]).wait()

    def drain128(r0, n):
        for q in range(0, n, K):
            m = min(K, n - q)
            pltpu.make_async_copy(hr[0].at[pl.ds(0, m)],
                                  out_sh.at[pl.ds(r0 + q, m)], isem[0]).wait()

    _rowcopy_split(s, denom_sh.shape[0], zero16)
    _rowcopy_split(s, out_sh.shape[0], zero128)
    _rowcopy_split(s, denom_sh.shape[0], drain16)
    _rowcopy_split(s, out_sh.shape[0], drain128)
    plsc.subcore_barrier()

    def issue_idx(j, b):
        pltpu.async_copy(src2_hbm.at[wid, j], srcv[b], isem[b])
        pltpu.async_copy(dst2_hbm.at[wid, j], dstv[b], isem[b])

    def wait_idx(j, b):
        pltpu.make_async_copy(src2_hbm.at[wid, j], srcv[b], isem[b]).wait()
        pltpu.make_async_copy(dst2_hbm.at[wid, j], dstv[b], isem[b]).wait()

    def issue_gathers(b):
        pltpu.async_copy(as_hbm.at[srcv[b]], rs[b], gsem[b])
        pltpu.async_copy(ad_hbm.at[dstv[b]], rd[b], gsem[b])
        pltpu.async_copy(h_hbm.at[srcv[b]], hr[b], gsem[b])

    def wait_gathers(b):
        pltpu.make_async_copy(as_hbm.at[srcv[b]], rs[b], gsem[b]).wait()
        pltpu.make_async_copy(ad_hbm.at[dstv[b]], rd[b], gsem[b]).wait()
        pltpu.make_async_copy(h_hbm.at[srcv[b]], hr[b], gsem[b]).wait()

    def issue_scatters(b):
        pltpu.async_copy(rs[b], denom_sh.at[dstv[b]], ssem[b], add=True)
        pltpu.async_copy(hr[b], out_sh.at[dstv[b]], ssem[b], add=True)

    def wait_scatters(b):
        pltpu.make_async_copy(rs[b], denom_sh.at[dstv[b]], ssem[b]).wait()
        pltpu.make_async_copy(hr[b], out_sh.at[dstv[b]], ssem[b]).wait()

    def compute(b):
        def edge_m(i, acc):
            a = rs[b][i, :] + rd[b][i, :]
            se = jnp.exp(jnp.maximum(a, NEG_SLOPE * a))
            rs[b][i, :] = se
            for hd in range(H):
                coefv = lax.gather(
                    se, jnp.full((16, 1), hd, dtype=jnp.int32),
                    lax.GatherDimensionNumbers(offset_dims=(),
                                               collapsed_slice_dims=(0,),
                                               start_index_map=(0,)),
                    slice_sizes=(1,),
                    mode=lax.GatherScatterMode.PROMISE_IN_BOUNDS)
                hr[b][i, pl.ds(hd * 16, 16)] = hr[b][i, pl.ds(hd * 16, 16)] * coefv
            return acc
        lax.fori_loop(0, K, edge_m, 0)

    IXL = 3
    GL = 2
    for j in range(IXL):
        issue_idx(j, j % NBUF)
    for j in range(GL):
        wait_idx(j, j % NBUF)
        issue_gathers(j % NBUF)

    def group(i, acc):
        for b in range(NBUF):
            j = i * NBUF + b
            if b >= GL:
                wait_scatters(b - GL)
            else:
                @pl.when(i >= 1)
                def _():
                    wait_scatters((b - GL) % NBUF)

            @pl.when(j + IXL < nch)
            def _():
                issue_idx(j + IXL, (b + IXL) % NBUF)

            @pl.when(j + GL < nch)
            def _():
                wait_idx(j + GL, (b + GL) % NBUF)
                issue_gathers((b + GL) % NBUF)
            wait_gathers(b)
            compute(b)
            issue_scatters(b)
        return acc
    lax.fori_loop(0, nch // NBUF, group, 0)

    for j in range(nch - GL, nch):
        wait_scatters(j % NBUF)

    plsc.subcore_barrier()
    _rowcopy_split(s, denom_sh.shape[0], lambda r0, n: pltpu.sync_copy(
        denom_sh.at[pl.ds(r0, n)], dpart.at[c, pl.ds(r0, n)]))
    _rowcopy_split(s, out_sh.shape[0], lambda r0, n: pltpu.sync_copy(
        out_sh.at[pl.ds(r0, n)], raw.at[c, pl.ds(r0, n)]))


def kernel(x, edge_index, W, att_src, att_dst, bias):
    N, D = x.shape
    E = edge_index.shape[1]
    HC = H * C
    nch = E // (NW * K)
    src2 = edge_index[0].astype(jnp.int32).reshape(NW, nch, K)
    dst2 = edge_index[1].astype(jnp.int32).reshape(NW, nch, K)

    asf = att_src.reshape(HC)
    adf = att_dst.reshape(HC)
    head_of = (jnp.arange(HC, dtype=jnp.int32) // C)
    M = (head_of[:, None] == jnp.arange(H, dtype=jnp.int32)[None, :]).astype(jnp.float32)
    S = jnp.concatenate([M * asf[:, None], M * adf[:, None],
                         M * adf[:, None], M * asf[:, None]], axis=1)
    P = jnp.concatenate([M.T, jnp.zeros((H, HC), jnp.float32)], axis=0)

    BLK = 1000
    grid = (N // BLK,)

    h, a4 = pl.pallas_call(
        _mm_body,
        grid=grid,
        in_specs=[pl.BlockSpec((BLK, D), lambda i: (i, 0)),
                  pl.BlockSpec((D, HC), lambda i: (0, 0)),
                  pl.BlockSpec((D, 2 * C), lambda i: (0, 0))],
        out_specs=[pl.BlockSpec((BLK, HC), lambda i: (i, 0)),
                   pl.BlockSpec((BLK, 2 * C), lambda i: (i, 0))],
        out_shape=[jax.ShapeDtypeStruct((N, HC), jnp.float32),
                   jax.ShapeDtypeStruct((N, 2 * C), jnp.float32)],
    )(x, W, S)

    a_sd = a4[:, :16]
    a_ds = a4[:, 16:]

    mesh = plsc.VectorSubcoreMesh(core_axis_name="c", subcore_axis_name="s",
                                  num_cores=NC, num_subcores=NS)

    dpart, raw = pl.kernel(
        _edge_body,
        out_type=(jax.ShapeDtypeStruct((NC, N, 16), jnp.float32),
                  jax.ShapeDtypeStruct((NC, N, HC), jnp.float32)),
        mesh=mesh,
        compiler_params=pltpu.CompilerParams(use_tc_tiling_on_sc=False),
        scratch_types=[
            [pltpu.VMEM((K,), jnp.int32)] * NBUF,
            [pltpu.VMEM((K,), jnp.int32)] * NBUF,
            [pltpu.VMEM((K, 16), jnp.float32)] * NBUF,
            [pltpu.VMEM((K, 16), jnp.float32)] * NBUF,
            [pltpu.VMEM((K, HC), jnp.float32)] * NBUF,
            pltpu.VMEM_SHARED((N, 16), jnp.float32),
            pltpu.VMEM_SHARED((N, HC), jnp.float32),
            [pltpu.SemaphoreType.DMA] * NBUF,
            [pltpu.SemaphoreType.DMA] * NBUF,
            [pltpu.SemaphoreType.DMA] * NBUF,
        ],
    )(a_sd, a_ds, h, src2, dst2)

    bias2d = bias.reshape(1, HC)
    out = pl.pallas_call(
        _epi_body,
        grid=grid,
        in_specs=[pl.BlockSpec((1, BLK, HC), lambda i: (0, i, 0)),
                  pl.BlockSpec((1, BLK, HC), lambda i: (1, i, 0)),
                  pl.BlockSpec((1, BLK, C), lambda i: (0, i, 0)),
                  pl.BlockSpec((1, BLK, C), lambda i: (1, i, 0)),
                  pl.BlockSpec((C, HC), lambda i: (0, 0)),
                  pl.BlockSpec((1, HC), lambda i: (0, 0))],
        out_specs=pl.BlockSpec((BLK, HC), lambda i: (i, 0)),
        out_shape=jax.ShapeDtypeStruct((N, HC), jnp.float32),
    )(raw, raw, dpart, dpart, P, bias2d)
    return out
